# baseline (speedup 1.0000x reference)
import functools

import jax
import jax.numpy as jnp
from jax import lax
from jax.experimental import pallas as pl
from jax.experimental.pallas import tpu as pltpu
from jax.experimental.pallas import tpu_sc as plsc

N_NODES = 10000
N_EDGES = 320000
D = 128

NC = 2
NS = 16
NW = NC * NS
NP = 10240
ROWS_PER_TILE = NP // NS
C2 = 128
BCH = 16
NBLK = 10
NCH = NBLK * BCH
E_PAD = NS * NCH * C2
ZR = 64
DEPTH = 2

_mesh = plsc.VectorSubcoreMesh(core_axis_name="c", subcore_axis_name="s")


def _zero_rows(zbuf, n_rows, n_cols):
    z = jnp.zeros((16,), jnp.float32)
    per_row = n_cols // 16

    def body(i, _):
        zbuf[i // per_row, pl.ds((i % per_row) * 16, 16)] = z
        return 0

    lax.fori_loop(0, n_rows * per_row, body, 0)


def _zero_flat(zbuf, n):
    z = jnp.zeros((16,), jnp.float32)

    def body(i, _):
        zbuf[pl.ds(i * 16, 16)] = z
        return 0

    lax.fori_loop(0, n // 16, body, 0)


def _guarded(active, body):
    lax.fori_loop(0, active, lambda i, _: (body(), 0)[1], 0)


@functools.partial(
    pl.kernel,
    out_type=jax.ShapeDtypeStruct((NP,), jnp.float32),
    mesh=_mesh,
    scratch_types=[
        pltpu.VMEM_SHARED((NP,), jnp.float32),
        pltpu.VMEM((NCH, C2), jnp.int32),
        pltpu.VMEM((C2,), jnp.float32),
        pltpu.VMEM((ROWS_PER_TILE,), jnp.float32),
        pltpu.SemaphoreType.DMA,
    ],
)
def _deg_kernel(dst_hbm, ones_hbm, out_hbm, acc_sh, dst_all, ones_v, zbuf, sem):
    cid = lax.axis_index("c")
    sid = lax.axis_index("s")
    act = jnp.where(cid == 0, 1, 0)
    n_groups = act * (NCH // 8)

    def setup():
        pltpu.sync_copy(dst_hbm.at[sid], dst_all)
        pltpu.sync_copy(ones_hbm, ones_v)
        _zero_flat(zbuf, ROWS_PER_TILE)
        pltpu.sync_copy(zbuf,
                        acc_sh.at[pl.ds(sid * ROWS_PER_TILE, ROWS_PER_TILE)])

    _guarded(act, setup)
    plsc.subcore_barrier()

    def group(g, _):
        descs = [
            pltpu.async_copy(ones_v, acc_sh.at[dst_all.at[g * 8 + j]], sem,
                             add=True)
            for j in range(8)
        ]
        for d in descs:
            d.wait()
        return 0

    lax.fori_loop(0, n_groups, group, 0)
    plsc.subcore_barrier()

    def copyout():
        off = sid * ROWS_PER_TILE
        pltpu.sync_copy(acc_sh.at[pl.ds(off, ROWS_PER_TILE)],
                        out_hbm.at[pl.ds(off, ROWS_PER_TILE)])

    _guarded(act, copyout)


@functools.partial(
    pl.kernel,
    out_type=jax.ShapeDtypeStruct((NP, D), jnp.float32),
    mesh=_mesh,
    scratch_types=[
        pltpu.VMEM_SHARED((NP, D), jnp.float32),
        pltpu.VMEM((BCH, C2), jnp.int32),
        pltpu.VMEM((BCH, C2), jnp.int32),
        pltpu.VMEM((DEPTH, C2, D), jnp.float32),
        pltpu.VMEM((ZR, D), jnp.float32),
        pltpu.SemaphoreType.DMA,
        pltpu.SemaphoreType.DMA,
    ],
)
def _agg_kernel(src_hbm, dst_hbm, h_hbm, out_hbm,
                acc_sh, src_blk, dst_blk, rows, zbuf, gsem, ssem):
    cid = lax.axis_index("c")
    sid = lax.axis_index("s")
    act = jnp.where(cid == 0, 1, 0)
    nblk = act * NBLK
    row0 = sid * ROWS_PER_TILE

    def zero():
        _zero_rows(zbuf, ZR, D)
        zd = [pltpu.async_copy(zbuf, acc_sh.at[pl.ds(row0 + k * ZR, ZR)],
                               gsem)
              for k in range(ROWS_PER_TILE // ZR)]
        for d in zd:
            d.wait()

    _guarded(act, zero)
    plsc.subcore_barrier()

    def blk(b, _):
        pltpu.sync_copy(src_hbm.at[sid, b], src_blk)
        pltpu.sync_copy(dst_hbm.at[sid, b], dst_blk)
        gd = [None] * BCH
        sd = [None] * BCH
        gd[0] = pltpu.async_copy(h_hbm.at[src_blk.at[0]], rows.at[0], gsem)
        for jj in range(BCH):
            nxt = jj + 1
            if nxt < BCH:
                if nxt >= DEPTH:
                    sd[nxt - DEPTH].wait()
                gd[nxt] = pltpu.async_copy(
                    h_hbm.at[src_blk.at[nxt]], rows.at[nxt % DEPTH], gsem)
            gd[jj].wait()
            sd[jj] = pltpu.async_copy(
                rows.at[jj % DEPTH], acc_sh.at[dst_blk.at[jj]], ssem,
                add=True)
        for jj in range(BCH - DEPTH, BCH):
            sd[jj].wait()
        return 0

    lax.fori_loop(0, nblk, blk, 0)
    plsc.subcore_barrier()

    def copyout():
        pltpu.sync_copy(acc_sh.at[pl.ds(row0, ROWS_PER_TILE)],
                        out_hbm.at[pl.ds(row0, ROWS_PER_TILE)])

    _guarded(act, copyout)


@functools.partial(
    pl.kernel,
    out_type=jax.ShapeDtypeStruct((NP,), jnp.float32),
    mesh=_mesh,
    scratch_types=[
        pltpu.VMEM_SHARED((NP,), jnp.float32),
        pltpu.VMEM((NCH, C2), jnp.int32),
        pltpu.VMEM((NCH, C2), jnp.int32),
        pltpu.VMEM((BCH, C2), jnp.float32),
        pltpu.VMEM((ROWS_PER_TILE,), jnp.float32),
        pltpu.SemaphoreType.DMA,
    ],
)
def _agg1_kernel(src_hbm, dst_hbm, h_hbm, out_hbm,
                 acc_sh, src_all, dst_all, vals_blk, zbuf, sem):
    cid = lax.axis_index("c")
    sid = lax.axis_index("s")
    act = jnp.where(cid == 0, 1, 0)
    nblk = act * NBLK

    def setup():
        pltpu.sync_copy(src_hbm.at[sid], src_all)
        pltpu.sync_copy(dst_hbm.at[sid], dst_all)
        _zero_flat(zbuf, ROWS_PER_TILE)
        pltpu.sync_copy(zbuf,
                        acc_sh.at[pl.ds(sid * ROWS_PER_TILE, ROWS_PER_TILE)])

    _guarded(act, setup)
    plsc.subcore_barrier()

    def blk(b, _):
        base = b * BCH
        gd = [pltpu.async_copy(h_hbm.at[src_all.at[base + j]], vals_blk.at[j],
                               sem)
              for j in range(BCH)]
        for d in gd:
            d.wait()
        sd = [pltpu.async_copy(vals_blk.at[j], acc_sh.at[dst_all.at[base + j]],
                               sem, add=True)
              for j in range(BCH)]
        for d in sd:
            d.wait()
        return 0

    lax.fori_loop(0, nblk, blk, 0)
    plsc.subcore_barrier()

    def copyout():
        off = sid * ROWS_PER_TILE
        pltpu.sync_copy(acc_sh.at[pl.ds(off, ROWS_PER_TILE)],
                        out_hbm.at[pl.ds(off, ROWS_PER_TILE)])

    _guarded(act, copyout)


def _dinv_body(dp_ref, o_ref):
    deg = dp_ref[...] + 1.0
    o_ref[...] = lax.rsqrt(deg)


def _mm_body(x_ref, w_ref, o_ref):
    o_ref[...] = jnp.dot(x_ref[...], w_ref[...],
                         preferred_element_type=jnp.float32)


def _scale_body(h_ref, dcol_ref, o_ref):
    o_ref[...] = h_ref[...] * dcol_ref[...]


def _layer_body(a_ref, hp_ref, dcol_ref, b_ref, w_ref, o_ref):
    s = (a_ref[...] + hp_ref[...]) * dcol_ref[...] + b_ref[...]
    h = jnp.maximum(s, 0.0)
    o_ref[...] = jnp.dot(h, w_ref[...],
                         preferred_element_type=jnp.float32) * dcol_ref[...]


def _final_body(a_ref, hp_ref, dcol_ref, b_ref, o_ref):
    o_ref[...] = (a_ref[...] + hp_ref[...]) * dcol_ref[...] + b_ref[...]


def _tc(body, out_shape, *args):
    return pl.pallas_call(
        body, out_shape=jax.ShapeDtypeStruct(out_shape, jnp.float32))(*args)


def kernel(x, edge_index, W1, b1, W2, b2, W3, b3):
    src = edge_index[0].astype(jnp.int32)
    dst = edge_index[1].astype(jnp.int32)
    pad = E_PAD - N_EDGES
    src3 = jnp.concatenate([src, jnp.zeros((pad,), jnp.int32)]
                           ).reshape(NS, NCH, C2)
    dst3 = jnp.concatenate([dst, jnp.full((pad,), NP - 1, jnp.int32)]
                           ).reshape(NS, NCH, C2)
    ones = jnp.ones((C2,), jnp.float32)

    deg = _deg_kernel(dst3, ones)
    h1 = _tc(_mm_body, (N_NODES, D), x, W1)
    dinv2d = _tc(_dinv_body, (NP // D, D), deg.reshape(NP // D, D))
    dinv_col = dinv2d.reshape(NP)[:N_NODES].reshape(N_NODES, 1)

    b1r = b1.reshape(1, D)
    b2r = b2.reshape(1, D)
    b3r = b3.reshape(1, 1)

    src4 = src3.reshape(NS, NBLK, BCH, C2)
    dst4 = dst3.reshape(NS, NBLK, BCH, C2)

    h1p = _tc(_scale_body, (N_NODES, D), h1, dinv_col)
    agg1 = _agg_kernel(src4, dst4, h1p)
    h2p = _tc(_layer_body, (N_NODES, D),
              agg1[:N_NODES], h1p, dinv_col, b1r, W2)
    agg2 = _agg_kernel(src4, dst4, h2p)
    h3p = _tc(_layer_body, (N_NODES, 1),
              agg2[:N_NODES], h2p, dinv_col, b2r, W3)
    agg3 = _agg1_kernel(src3, dst3, h3p.reshape(N_NODES))
    out = _tc(_final_body, (N_NODES, 1),
              agg3[:N_NODES].reshape(N_NODES, 1),
              h3p, dinv_col, b3r)
    return out


# speedup vs baseline: 1.2885x; 1.2885x over previous
import functools

import jax
import jax.numpy as jnp
from jax import lax
from jax.experimental import pallas as pl
from jax.experimental.pallas import tpu as pltpu
from jax.experimental.pallas import tpu_sc as plsc

N_NODES = 10000
N_EDGES = 320000
D = 128

NC = 2
NS = 16
NW = NC * NS
NP = 10240
ROWS_PER_TILE = NP // NS
C2 = 128
BCH = 16
NBLK0 = 9
NBLK1 = 1
NCH0 = NBLK0 * BCH
NCH1 = NBLK1 * BCH
E0 = NS * NCH0 * C2
E1 = NS * NCH1 * C2
ZR = 64
DEPTH = 2

_mesh = plsc.VectorSubcoreMesh(core_axis_name="c", subcore_axis_name="s")


def _zero_rows(zbuf, n_rows, n_cols):
    z = jnp.zeros((16,), jnp.float32)
    per_row = n_cols // 16

    def body(i, _):
        zbuf[i // per_row, pl.ds((i % per_row) * 16, 16)] = z
        return 0

    lax.fori_loop(0, n_rows * per_row, body, 0)


def _zero_flat(zbuf, n):
    z = jnp.zeros((16,), jnp.float32)

    def body(i, _):
        zbuf[pl.ds(i * 16, 16)] = z
        return 0

    lax.fori_loop(0, n // 16, body, 0)


@functools.partial(
    pl.kernel,
    out_type=jax.ShapeDtypeStruct((NC * NP,), jnp.float32),
    mesh=_mesh,
    scratch_types=[
        pltpu.VMEM_SHARED((NP,), jnp.float32),
        pltpu.VMEM((NCH0, C2), jnp.int32),
        pltpu.VMEM((C2,), jnp.float32),
        pltpu.VMEM((ROWS_PER_TILE,), jnp.float32),
        pltpu.SemaphoreType.DMA,
    ],
)
def _deg_kernel(dst_hbm, ones_hbm, out_hbm, acc_sh, dst_all, ones_v, zbuf, sem):
    cid = lax.axis_index("c")
    sid = lax.axis_index("s")
    wid = cid * NS + sid
    n_groups = jnp.where(cid == 0, NCH0 // 8, NCH1 // 8)

    pltpu.sync_copy(dst_hbm.at[wid], dst_all)
    pltpu.sync_copy(ones_hbm, ones_v)
    _zero_flat(zbuf, ROWS_PER_TILE)
    pltpu.sync_copy(zbuf, acc_sh.at[pl.ds(sid * ROWS_PER_TILE, ROWS_PER_TILE)])
    plsc.subcore_barrier()

    def group(g, _):
        descs = [
            pltpu.async_copy(ones_v, acc_sh.at[dst_all.at[g * 8 + j]], sem,
                             add=True)
            for j in range(8)
        ]
        for d in descs:
            d.wait()
        return 0

    lax.fori_loop(0, n_groups, group, 0)
    plsc.subcore_barrier()
    off = sid * ROWS_PER_TILE
    pltpu.sync_copy(acc_sh.at[pl.ds(off, ROWS_PER_TILE)],
                    out_hbm.at[pl.ds(cid * NP + off, ROWS_PER_TILE)])


@functools.partial(
    pl.kernel,
    out_type=jax.ShapeDtypeStruct((NC * NP, D), jnp.float32),
    mesh=_mesh,
    scratch_types=[
        pltpu.VMEM_SHARED((NP, D), jnp.float32),
        pltpu.VMEM((BCH, C2), jnp.int32),
        pltpu.VMEM((BCH, C2), jnp.int32),
        pltpu.VMEM((DEPTH, C2, D), jnp.float32),
        pltpu.VMEM((ZR, D), jnp.float32),
        pltpu.SemaphoreType.DMA,
        pltpu.SemaphoreType.DMA,
    ],
)
def _agg_kernel(src_hbm, dst_hbm, h_hbm, out_hbm,
                acc_sh, src_blk, dst_blk, rows, zbuf, gsem, ssem):
    cid = lax.axis_index("c")
    sid = lax.axis_index("s")
    wid = cid * NS + sid
    nblk = jnp.where(cid == 0, NBLK0, NBLK1)

    _zero_rows(zbuf, ZR, D)
    row0 = sid * ROWS_PER_TILE
    zd = [pltpu.async_copy(zbuf, acc_sh.at[pl.ds(row0 + k * ZR, ZR)], gsem)
          for k in range(ROWS_PER_TILE // ZR)]
    for d in zd:
        d.wait()
    plsc.subcore_barrier()

    def blk(b, _):
        pltpu.sync_copy(src_hbm.at[wid, b], src_blk)
        pltpu.sync_copy(dst_hbm.at[wid, b], dst_blk)
        gd = [None] * BCH
        sd = [None] * BCH
        gd[0] = pltpu.async_copy(h_hbm.at[src_blk.at[0]], rows.at[0], gsem)
        for jj in range(BCH):
            nxt = jj + 1
            if nxt < BCH:
                if nxt >= DEPTH:
                    sd[nxt - DEPTH].wait()
                gd[nxt] = pltpu.async_copy(
                    h_hbm.at[src_blk.at[nxt]], rows.at[nxt % DEPTH], gsem)
            gd[jj].wait()
            sd[jj] = pltpu.async_copy(
                rows.at[jj % DEPTH], acc_sh.at[dst_blk.at[jj]], ssem,
                add=True)
        for jj in range(BCH - DEPTH, BCH):
            sd[jj].wait()
        return 0

    lax.fori_loop(0, nblk, blk, 0)
    plsc.subcore_barrier()
    pltpu.sync_copy(acc_sh.at[pl.ds(row0, ROWS_PER_TILE)],
                    out_hbm.at[pl.ds(cid * NP + row0, ROWS_PER_TILE)])


@functools.partial(
    pl.kernel,
    out_type=jax.ShapeDtypeStruct((NC * NP,), jnp.float32),
    mesh=_mesh,
    scratch_types=[
        pltpu.VMEM_SHARED((NP,), jnp.float32),
        pltpu.VMEM((NCH0, C2), jnp.int32),
        pltpu.VMEM((NCH0, C2), jnp.int32),
        pltpu.VMEM((BCH, C2), jnp.float32),
        pltpu.VMEM((ROWS_PER_TILE,), jnp.float32),
        pltpu.SemaphoreType.DMA,
    ],
)
def _agg1_kernel(src_hbm, dst_hbm, h_hbm, out_hbm,
                 acc_sh, src_all, dst_all, vals_blk, zbuf, sem):
    cid = lax.axis_index("c")
    sid = lax.axis_index("s")
    wid = cid * NS + sid
    nblk = jnp.where(cid == 0, NBLK0, NBLK1)

    pltpu.sync_copy(src_hbm.at[wid], src_all)
    pltpu.sync_copy(dst_hbm.at[wid], dst_all)
    _zero_flat(zbuf, ROWS_PER_TILE)
    pltpu.sync_copy(zbuf, acc_sh.at[pl.ds(sid * ROWS_PER_TILE, ROWS_PER_TILE)])
    plsc.subcore_barrier()

    def blk(b, _):
        base = b * BCH
        gd = [pltpu.async_copy(h_hbm.at[src_all.at[base + j]], vals_blk.at[j],
                               sem)
              for j in range(BCH)]
        for d in gd:
            d.wait()
        sd = [pltpu.async_copy(vals_blk.at[j], acc_sh.at[dst_all.at[base + j]],
                               sem, add=True)
              for j in range(BCH)]
        for d in sd:
            d.wait()
        return 0

    lax.fori_loop(0, nblk, blk, 0)
    plsc.subcore_barrier()
    off = sid * ROWS_PER_TILE
    pltpu.sync_copy(acc_sh.at[pl.ds(off, ROWS_PER_TILE)],
                    out_hbm.at[pl.ds(cid * NP + off, ROWS_PER_TILE)])


def _dinv_body(dp_ref, o_ref):
    deg = dp_ref[0] + dp_ref[1] + 1.0
    o_ref[...] = lax.rsqrt(deg)


def _mm_body(x_ref, w_ref, o_ref):
    o_ref[...] = jnp.dot(x_ref[...], w_ref[...],
                         preferred_element_type=jnp.float32)


def _scale_body(h_ref, dcol_ref, o_ref):
    o_ref[...] = h_ref[...] * dcol_ref[...]


def _layer_body(a0_ref, a1_ref, hp_ref, dcol_ref, b_ref, w_ref, o_ref):
    s = (a0_ref[...] + a1_ref[...] + hp_ref[...]) * dcol_ref[...] + b_ref[...]
    h = jnp.maximum(s, 0.0)
    o_ref[...] = jnp.dot(h, w_ref[...],
                         preferred_element_type=jnp.float32) * dcol_ref[...]


def _final_body(a0_ref, a1_ref, hp_ref, dcol_ref, b_ref, o_ref):
    o_ref[...] = ((a0_ref[...] + a1_ref[...] + hp_ref[...]) * dcol_ref[...]
                  + b_ref[...])


def _tc(body, out_shape, *args):
    return pl.pallas_call(
        body, out_shape=jax.ShapeDtypeStruct(out_shape, jnp.float32))(*args)


def kernel(x, edge_index, W1, b1, W2, b2, W3, b3):
    src = edge_index[0].astype(jnp.int32)
    dst = edge_index[1].astype(jnp.int32)

    def layout(e, fill):
        fast = e[:E0].reshape(NS, NCH0, C2)
        pad = E1 - (N_EDGES - E0)
        slow = jnp.concatenate([e[E0:], jnp.full((pad,), fill, jnp.int32)]
                               ).reshape(NS, NCH1, C2)
        cap = jnp.full((NS, NCH0 - NCH1, C2), fill, jnp.int32)
        return jnp.concatenate(
            [fast, jnp.concatenate([slow, cap], axis=1)]).reshape(
                NW, NCH0, C2)

    src3 = layout(src, 0)
    dst3 = layout(dst, NP - 1)
    ones = jnp.ones((C2,), jnp.float32)

    deg_parts = _deg_kernel(dst3, ones)
    h1 = _tc(_mm_body, (N_NODES, D), x, W1)
    dinv2d = _tc(_dinv_body, (NP // D, D), deg_parts.reshape(NC, NP // D, D))
    dinv_col = dinv2d.reshape(NP)[:N_NODES].reshape(N_NODES, 1)

    b1r = b1.reshape(1, D)
    b2r = b2.reshape(1, D)
    b3r = b3.reshape(1, 1)

    src4 = src3.reshape(NW, NBLK0, BCH, C2)
    dst4 = dst3.reshape(NW, NBLK0, BCH, C2)

    h1p = _tc(_scale_body, (N_NODES, D), h1, dinv_col)
    agg1 = _agg_kernel(src4, dst4, h1p).reshape(NC, NP, D)
    h2p = _tc(_layer_body, (N_NODES, D),
              agg1[0, :N_NODES], agg1[1, :N_NODES], h1p, dinv_col, b1r, W2)
    agg2 = _agg_kernel(src4, dst4, h2p).reshape(NC, NP, D)
    h3p = _tc(_layer_body, (N_NODES, 1),
              agg2[0, :N_NODES], agg2[1, :N_NODES], h2p, dinv_col, b2r, W3)
    agg3 = _agg1_kernel(src3, dst3, h3p.reshape(N_NODES)).reshape(NC, NP)
    out = _tc(_final_body, (N_NODES, 1),
              agg3[0, :N_NODES].reshape(N_NODES, 1),
              agg3[1, :N_NODES].reshape(N_NODES, 1),
              h3p, dinv_col, b3r)
    return out
